# baseline (speedup 1.0000x reference)
import jax
import jax.numpy as jnp
from jax.experimental import pallas as pl
from jax.experimental.pallas import tpu as pltpu

N = 10000
D = 128
CH = 400
NC = N // CH
HALF = CH // 2


def _leaky(x):
    return jnp.where(x >= 0, x, 0.01 * x)


def _body(ego_hbm, a_hbm, wgc_ref, bgc_ref, wbi_ref, bbi_ref, out_hbm,
          egob, abuf0, abuf1, outb0, outb1, sems, osems):
    wgc = wgc_ref[...]
    bgc = bgc_ref[...]
    wbi = wbi_ref[...]
    bbi = bbi_ref[...]

    def epilogue(nb, ego):
        add = jax.lax.dot_general(ego + nb, wgc, (((1,), (1,)), ((), ())),
                                  preferred_element_type=jnp.float32)
        wise = jax.lax.dot_general(ego * nb, wbi, (((1,), (1,)), ((), ())),
                                   preferred_element_type=jnp.float32)
        return _leaky(add + bgc) + _leaky(wise + bbi)

    ego_cp = pltpu.make_async_copy(ego_hbm, egob, sems.at[0])
    ego_cp.start()
    h0 = pltpu.make_async_copy(a_hbm.at[pl.ds(0, HALF)],
                               abuf0.at[pl.ds(0, HALF)], sems.at[1])
    h1 = pltpu.make_async_copy(a_hbm.at[pl.ds(HALF, HALF)],
                               abuf0.at[pl.ds(HALF, HALF)], sems.at[2])
    h0.start()
    h1.start()
    c1 = pltpu.make_async_copy(a_hbm.at[pl.ds(CH, CH)], abuf1, sems.at[3])
    c1.start()

    bufs = (abuf0, abuf1)
    outbs = (outb0, outb1)

    ego_cp.wait()
    h0.wait()
    nb0 = jnp.dot(abuf0[pl.ds(0, HALF), :], egob[...],
                  preferred_element_type=jnp.float32)
    r0 = epilogue(nb0, egob[pl.ds(0, HALF), :])
    h1.wait()
    nb1 = jnp.dot(abuf0[pl.ds(HALF, HALF), :], egob[...],
                  preferred_element_type=jnp.float32)
    r1 = epilogue(nb1, egob[pl.ds(HALF, HALF), :])
    outb0[pl.ds(0, HALF), :] = r0
    outb0[pl.ds(HALF, HALF), :] = r1
    pltpu.make_async_copy(outb0, out_hbm.at[pl.ds(0, CH)], osems.at[0]).start()

    for c in range(1, NC):
        slot = c % 2
        if c + 1 < NC:
            pltpu.make_async_copy(a_hbm.at[pl.ds((c + 1) * CH, CH)],
                                  bufs[(c + 1) % 2],
                                  sems.at[3 + c % 2]).start()
        pltpu.make_async_copy(a_hbm.at[pl.ds(c * CH, CH)], bufs[slot],
                              sems.at[3 + (c - 1) % 2]).wait()
        nb = jnp.dot(bufs[slot][...], egob[...],
                     preferred_element_type=jnp.float32)
        res = epilogue(nb, egob[pl.ds(c * CH, CH), :])
        if c >= 2:
            pltpu.make_async_copy(outbs[slot],
                                  out_hbm.at[pl.ds((c - 2) * CH, CH)],
                                  osems.at[slot]).wait()
        outbs[slot][...] = res
        pltpu.make_async_copy(outbs[slot], out_hbm.at[pl.ds(c * CH, CH)],
                              osems.at[slot]).start()

    pltpu.make_async_copy(outbs[(NC - 1) % 2],
                          out_hbm.at[pl.ds((NC - 1) * CH, CH)],
                          osems.at[(NC - 1) % 2]).wait()
    pltpu.make_async_copy(outbs[(NC - 2) % 2],
                          out_hbm.at[pl.ds((NC - 2) * CH, CH)],
                          osems.at[(NC - 2) % 2]).wait()


@jax.jit
def kernel(ego_embed, A_in, W_gc, b_gc, W_bi, b_bi):
    hbm = pl.BlockSpec(memory_space=pltpu.MemorySpace.HBM)
    vmem = pl.BlockSpec(memory_space=pltpu.MemorySpace.VMEM)
    return pl.pallas_call(
        _body,
        in_specs=[hbm, hbm, vmem, vmem, vmem, vmem],
        out_specs=hbm,
        out_shape=jax.ShapeDtypeStruct((N, D), jnp.float32),
        scratch_shapes=[
            pltpu.VMEM((N, D), jnp.float32),
            pltpu.VMEM((CH, N), jnp.float32),
            pltpu.VMEM((CH, N), jnp.float32),
            pltpu.VMEM((CH, D), jnp.float32),
            pltpu.VMEM((CH, D), jnp.float32),
            pltpu.SemaphoreType.DMA((5,)),
            pltpu.SemaphoreType.DMA((2,)),
        ],
    )(ego_embed, A_in, W_gc, b_gc.reshape(1, D), W_bi, b_bi.reshape(1, D))
